# Initial kernel scaffold; baseline (speedup 1.0000x reference)
import jax
import jax.numpy as jnp
from jax.experimental import pallas as pl


def kernel(x, edge_index, edge_attr, batch, Wt1_1, bt1_1, Wg1, bg1, Wt2_1, bt2_1, Wt1_2, bt1_2, Wg2, bg2, Wt2_2, bt2_2, Wc, bc, Wf, bf):
    raise NotImplementedError("write your pallas kernel here")


# speedup vs baseline: 9.1334x; 9.1334x over previous
import functools

import jax
import jax.numpy as jnp
from jax import lax
from jax.experimental import pallas as pl
from jax.experimental.pallas import tpu as pltpu
from jax.experimental.pallas import tpu_sc as plsc

BS = 128
EC = 32
N = BS * EC
E = 65536
WC = 60
CS = 128
K = 8

G = 2
NPB = G * EC
NBLK = BS // G
ABD = NBLK * NPB * NPB

T1 = WC - K + 1
T2 = T1 - K + 1
T3 = T2 - K + 1
T4 = T3 - K + 1


def _build_adj(src2, dst2, ew2, zeros_hbm):
    mesh = plsc.VectorSubcoreMesh(core_axis_name="c", subcore_axis_name="s")
    rows_per_worker = 16
    words_per_tile = ABD // 16

    @functools.partial(
        pl.kernel,
        mesh=mesh,
        out_type=jax.ShapeDtypeStruct((2, ABD), jnp.float32),
        scratch_types=[
            pltpu.VMEM((rows_per_worker, 128), jnp.int32),
            pltpu.VMEM((rows_per_worker, 128), jnp.int32),
            pltpu.VMEM((rows_per_worker, 128), jnp.float32),
            pltpu.VMEM((rows_per_worker, 128), jnp.int32),
            pltpu.VMEM_SHARED((ABD,), jnp.float32),
        ],
    )
    def adj(src_hbm, dst_hbm, ew_hbm, zero_hbm, out_hbm,
            src_v, dst_v, ew_v, idx_v, a_sh):
        cid = lax.axis_index("c")
        sid = lax.axis_index("s")
        pltpu.sync_copy(zero_hbm.at[pl.ds(sid * words_per_tile, words_per_tile)],
                        a_sh.at[pl.ds(sid * words_per_tile, words_per_tile)])
        plsc.subcore_barrier()
        rbase = (cid * 16 + sid) * rows_per_worker
        pltpu.sync_copy(src_hbm.at[pl.ds(rbase, rows_per_worker), :], src_v)
        pltpu.sync_copy(dst_hbm.at[pl.ds(rbase, rows_per_worker), :], dst_v)
        pltpu.sync_copy(ew_hbm.at[pl.ds(rbase, rows_per_worker), :], ew_v)
        for j in range(rows_per_worker):
            def body(i, carry, j=j):
                s16 = src_v[j, pl.ds(i * 16, 16)]
                d16 = dst_v[j, pl.ds(i * 16, 16)]
                blk = lax.shift_right_logical(d16, NPB.bit_length() - 1)
                idx16 = blk * (NPB * NPB) + (d16 & (NPB - 1)) * NPB + (s16 & (NPB - 1))
                idx_v[j, pl.ds(i * 16, 16)] = idx16
                return carry
            lax.fori_loop(0, 128 // 16, body, 0)
        for j in range(rows_per_worker):
            pltpu.sync_copy(ew_v.at[j], a_sh.at[idx_v.at[j]], add=True)
        plsc.subcore_barrier()
        pltpu.sync_copy(a_sh.at[pl.ds(sid * words_per_tile, words_per_tile)],
                        out_hbm.at[cid, pl.ds(sid * words_per_tile, words_per_tile)])

    return adj(src2, dst2, ew2, zeros_hbm)


def _gcn_block(h, a1, wg, bg, t_len, c):
    f32 = jnp.float32
    aggs = [jnp.dot(a1, h[:, t, :], preferred_element_type=f32)[:, None, :]
            for t in range(t_len)]
    agg = jnp.concatenate(aggs, axis=1)
    hw = jnp.dot(agg.reshape(NPB * t_len, c), wg, preferred_element_type=f32)
    return jnp.maximum(hw.reshape(NPB, t_len, c) + bg[None, None, :], 0.0)


def _tc_body(x_ref, a_ref, w1_ref, b1_ref, wg1_ref, bg1_ref, w2_ref, b2_ref,
             w3_ref, b3_ref, wg2_ref, bg2_ref, w4_ref, b4_ref, wc_ref, bc_ref,
             wf_ref, bf_ref, out_ref):
    f32 = jnp.float32
    rows = lax.broadcasted_iota(jnp.int32, (NPB, NPB), 0)
    cols = lax.broadcasted_iota(jnp.int32, (NPB, NPB), 1)
    a1 = a_ref[0, 0] + a_ref[1, 0] + jnp.where(rows == cols, 1.0, 0.0).astype(f32)

    h = x_ref[:].reshape(NPB * WC, CS)
    y = jnp.dot(h, w1_ref[:], preferred_element_type=f32).reshape(NPB, WC, K * 32)
    h1 = y[:, 0:T1, 0:32]
    for k in range(1, K):
        h1 = h1 + y[:, k:k + T1, k * 32:(k + 1) * 32]
    h1 = jnp.maximum(h1 + b1_ref[:][None, None, :], 0.0)

    z1 = _gcn_block(h1, a1, wg1_ref[:], bg1_ref[:], T1, 32)

    c2 = jnp.concatenate([z1[:, k:k + T2, :] for k in range(K)], axis=2)
    y2 = jnp.dot(c2.reshape(NPB * T2, K * 32), w2_ref[:], preferred_element_type=f32)
    h2 = jnp.maximum(y2.reshape(NPB, T2, 64) + b2_ref[:][None, None, :], 0.0)

    y3 = jnp.dot(h2.reshape(NPB * T2, 64), w3_ref[:], preferred_element_type=f32)
    y3 = y3.reshape(NPB, T2, K * 16)
    h3 = y3[:, 0:T3, 0:16]
    for k in range(1, K):
        h3 = h3 + y3[:, k:k + T3, k * 16:(k + 1) * 16]
    h3 = jnp.maximum(h3 + b3_ref[:][None, None, :], 0.0)

    z2 = _gcn_block(h3, a1, wg2_ref[:], bg2_ref[:], T3, 16)

    c4 = jnp.concatenate([z2[:, k:k + T4, :] for k in range(K)], axis=2)
    y4 = jnp.dot(c4.reshape(NPB * T4, K * 16), w4_ref[:], preferred_element_type=f32)
    h4 = jnp.maximum(y4.reshape(NPB, T4, 32) + b4_ref[:][None, None, :], 0.0)

    y5 = jnp.dot(h4.reshape(NPB, T4 * 32), wc_ref[:], preferred_element_type=f32)
    y5 = y5 + bc_ref[:][None, :]

    prod = y5 * wf_ref[:]
    grow = lax.broadcasted_iota(jnp.int32, (G, NPB), 0)
    gcol = lax.broadcasted_iota(jnp.int32, (G, NPB), 1)
    gsel = jnp.where(gcol // EC == grow, 1.0, 0.0).astype(f32)
    per_graph = jnp.dot(gsel, prod, preferred_element_type=f32)
    o = jnp.dot(per_graph, jnp.ones((32, 1), f32),
                preferred_element_type=f32) + bf_ref[:][None, :]
    out_ref[:] = jnp.maximum(o, 0.0).reshape(1, G, 1)


def _stgcn_tc(x, a2, w1c, b1, wg1, bg1, w2i, b2, w3c, b3, wg2, bg2, w4i, b4,
              wct, bc, wft, bf):
    full = lambda shape: pl.BlockSpec(shape, lambda i: (0,) * len(shape))
    out = pl.pallas_call(
        _tc_body,
        grid=(NBLK,),
        in_specs=[
            pl.BlockSpec((NPB, WC * CS), lambda i: (i, 0)),
            pl.BlockSpec((2, 1, NPB, NPB), lambda i: (0, i, 0, 0)),
            full((CS, K * 32)), full((32,)),
            full((32, 32)), full((32,)),
            full((K * 32, 64)), full((64,)),
            full((64, K * 16)), full((16,)),
            full((16, 16)), full((16,)),
            full((K * 16, 32)), full((32,)),
            full((T4 * 32, 32)), full((32,)),
            full((NPB, 32)), full((1,)),
        ],
        out_specs=pl.BlockSpec((1, G, 1), lambda i: (i, 0, 0)),
        out_shape=jax.ShapeDtypeStruct((NBLK, G, 1), jnp.float32),
    )(x, a2, w1c, b1, wg1, bg1, w2i, b2, w3c, b3, wg2, bg2, w4i, b4,
      wct, bc, wft, bf)
    return out.reshape(BS, 1)


def kernel(x, edge_index, edge_attr, batch, Wt1_1, bt1_1, Wg1, bg1, Wt2_1,
           bt2_1, Wt1_2, bt1_2, Wg2, bg2, Wt2_2, bt2_2, Wc, bc, Wf, bf):
    src2 = edge_index[0].reshape(E // 128, 128)
    dst2 = edge_index[1].reshape(E // 128, 128)
    ew2 = edge_attr[:, 0].reshape(E // 128, 128)
    zeros_hbm = jnp.zeros((ABD,), jnp.float32)
    a_part = _build_adj(src2, dst2, ew2, zeros_hbm)
    a2 = a_part.reshape(2, NBLK, NPB, NPB)

    w1c = Wt1_1.transpose(1, 2, 0).reshape(CS, K * 32)
    w2i = Wt2_1.transpose(2, 1, 0).reshape(K * 32, 64)
    w3c = Wt1_2.transpose(1, 2, 0).reshape(64, K * 16)
    w4i = Wt2_2.transpose(2, 1, 0).reshape(K * 16, 32)
    wct = Wc.transpose(2, 1, 0).reshape(T4 * 32, 32)
    wft = jnp.tile(Wf.reshape(EC, 32), (G, 1))

    return _stgcn_tc(x, a2, w1c, bt1_1, Wg1, bg1, w2i, bt2_1, w3c, bt1_2,
                     Wg2, bg2, w4i, bt2_2, wct, bc, wft, bf)
